# baseline (speedup 1.0000x reference)
import functools

import jax
import jax.numpy as jnp
from jax import lax
from jax.experimental import pallas as pl
from jax.experimental.pallas import tpu as pltpu
from jax.experimental.pallas import tpu_sc as plsc

NP_N = 10000
NA_N = 10000
NS_N = 512
D = 128

_NC = 2
_NSUB = 16
_NW = _NC * _NSUB
_L = 16
CH = 64
ACC_R = 10240
SPAN_MAX = 80


def _node_tc_kernel(x_ref, w1_ref, w2_ref, w3_ref, w0_ref, b_ref, av_ref,
                    z1_ref, z2_ref, z3_ref, s_ref):
    x = x_ref[...]
    dn = (((1,), (1,)), ((), ()))
    z1 = lax.dot_general(x, w1_ref[...], dn, preferred_element_type=jnp.float32)
    z2 = lax.dot_general(x, w2_ref[...], dn, preferred_element_type=jnp.float32)
    z3 = lax.dot_general(x, w3_ref[...], dn, preferred_element_type=jnp.float32)
    z0 = lax.dot_general(x, w0_ref[...], dn, preferred_element_type=jnp.float32)
    b = b_ref[...]
    z1 = z1 + b[0:1, :]
    z2 = z2 + b[1:2, :]
    z3 = z3 + b[2:3, :]
    z0 = z0 + b[3:4, :]
    av = av_ref[...]
    s1 = lax.dot_general(z1, av[0:1, :], dn, preferred_element_type=jnp.float32)
    s2 = lax.dot_general(z2, av[1:2, :], dn, preferred_element_type=jnp.float32)
    s3 = lax.dot_general(z3, av[2:3, :], dn, preferred_element_type=jnp.float32)
    d1 = lax.dot_general(z0, av[3:4, :], dn, preferred_element_type=jnp.float32)
    d2 = lax.dot_general(z0, av[4:5, :], dn, preferred_element_type=jnp.float32)
    z1_ref[...] = z1
    z2_ref[...] = z2
    z3_ref[...] = z3
    s_ref[...] = jnp.concatenate(
        [s1, s2, s3, d1, d2, jnp.zeros_like(s1), jnp.zeros_like(s1),
         jnp.zeros_like(s1)], axis=1)


def _node_transform(x, w1, w2, w3, w0, bmat, av, block_rows):
    n = x.shape[0]
    z_spec = pl.BlockSpec((block_rows, D), lambda i: (i, 0))
    return pl.pallas_call(
        _node_tc_kernel,
        grid=(n // block_rows,),
        in_specs=[
            pl.BlockSpec((block_rows, D), lambda i: (i, 0)),
            pl.BlockSpec((D, D), lambda i: (0, 0)),
            pl.BlockSpec((D, D), lambda i: (0, 0)),
            pl.BlockSpec((D, D), lambda i: (0, 0)),
            pl.BlockSpec((D, D), lambda i: (0, 0)),
            pl.BlockSpec((4, D), lambda i: (0, 0)),
            pl.BlockSpec((8, D), lambda i: (0, 0)),
        ],
        out_specs=[z_spec, z_spec, z_spec,
                   pl.BlockSpec((block_rows, 8), lambda i: (i, 0))],
        out_shape=[
            jax.ShapeDtypeStruct((n, D), jnp.float32),
            jax.ShapeDtypeStruct((n, D), jnp.float32),
            jax.ShapeDtypeStruct((n, D), jnp.float32),
            jax.ShapeDtypeStruct((n, 8), jnp.float32),
        ],
    )(x, w1, w2, w3, w0, bmat, av)


def _state_tc_kernel(x_ref, w0_ref, b_ref, av_ref, z_ref, s_ref):
    x = x_ref[...]
    dn = (((1,), (1,)), ((), ()))
    z0 = lax.dot_general(x, w0_ref[...], dn, preferred_element_type=jnp.float32)
    z0 = z0 + b_ref[...][0:1, :]
    av = av_ref[...]
    d1 = lax.dot_general(z0, av[0:1, :], dn, preferred_element_type=jnp.float32)
    d2 = lax.dot_general(z0, av[1:2, :], dn, preferred_element_type=jnp.float32)
    z_ref[...] = z0
    s_ref[...] = jnp.concatenate([d1, d2] + [jnp.zeros_like(d1)] * 6, axis=1)


def _state_transform(x, w0, b, av):
    n = x.shape[0]
    return pl.pallas_call(
        _state_tc_kernel,
        out_shape=[
            jax.ShapeDtypeStruct((n, D), jnp.float32),
            jax.ShapeDtypeStruct((n, 8), jnp.float32),
        ],
    )(x, w0, b.reshape(1, D), av)


def _combine_pa_kernel(a_ref, s_ref, o_ref):
    br = a_ref.shape[2]
    i = pl.program_id(0)
    a = a_ref[...]
    nr = br // 128
    sl = pl.ds(i * nr, nr)
    s00 = s_ref[0, 0, sl, :].reshape(br)
    s01 = s_ref[0, 1, sl, :].reshape(br)
    s10 = s_ref[1, 0, sl, :].reshape(br)
    s11 = s_ref[1, 1, sl, :].reshape(br)
    o = ((a[0, 0] + a[0, 1]) / (s00 + s01 + 1e-16)[:, None]
         + (a[1, 0] + a[1, 1]) / (s10 + s11 + 1e-16)[:, None])
    o_ref[...] = jnp.maximum(o, 0.0)


def _combine_pa(acc, ssum, n_out, block_rows):
    ssum_r = ssum.reshape(2, 2, ACC_R // 128, 128)
    full = pl.pallas_call(
        _combine_pa_kernel,
        grid=(ACC_R // block_rows,),
        in_specs=[
            pl.BlockSpec((2, 2, block_rows, D), lambda i: (0, 0, i, 0)),
            pl.BlockSpec((2, 2, ACC_R // 128, 128), lambda i: (0, 0, 0, 0)),
        ],
        out_specs=pl.BlockSpec((block_rows, D), lambda i: (i, 0)),
        out_shape=jax.ShapeDtypeStruct((ACC_R, D), jnp.float32),
    )(acc, ssum_r)
    return full[:n_out]


def _combine_s_kernel(a_ref, s_ref, o_ref):
    a = a_ref[...]
    s = s_ref[...]
    o = ((a[0, 0] + a[0, 1]) / (s[0, 0] + s[0, 1] + 1e-16)[:, None]
         + (a[1, 0] + a[1, 1]) / (s[1, 0] + s[1, 1] + 1e-16)[:, None]
         + a[2, 0] + a[2, 1])
    o_ref[...] = jnp.maximum(o, 0.0)


def _combine_s(acc, ssum):
    return pl.pallas_call(
        _combine_s_kernel,
        out_shape=jax.ShapeDtypeStruct((NS_N, D), jnp.float32),
    )(acc, ssum)


def _pad_edge(e32, pad_dst, nk_mult):
    n_e = e32.shape[1]
    nk = -(-n_e // CH)
    nk = -(-nk // nk_mult) * nk_mult
    pad = nk * CH - n_e
    src = jnp.concatenate([e32[0], jnp.zeros((pad,), jnp.int32)])
    dst = jnp.concatenate([e32[1], jnp.full((pad,), pad_dst, jnp.int32)])
    return src.reshape(nk, CH), dst.reshape(nk, CH), nk


def _edge_sc(zp1, zp2, zp3, za1, za2, za3, zs, spt, sat, sst, edges):
    nk_of = {r: edges[r][2] for r in edges}
    mesh = plsc.VectorSubcoreMesh(core_axis_name="c", subcore_axis_name="s",
                                  num_cores=_NC, num_subcores=_NSUB)

    @functools.partial(
        pl.kernel,
        out_type=[
            jax.ShapeDtypeStruct((2, _NC, ACC_R, D), jnp.float32),
            jax.ShapeDtypeStruct((2, _NC, ACC_R, D), jnp.float32),
            jax.ShapeDtypeStruct((3, _NC, NS_N, D), jnp.float32),
            jax.ShapeDtypeStruct((2, _NC, ACC_R), jnp.float32),
            jax.ShapeDtypeStruct((2, _NC, ACC_R), jnp.float32),
            jax.ShapeDtypeStruct((2, _NC, NS_N), jnp.float32),
        ],
        mesh=mesh,
        compiler_params=pltpu.CompilerParams(needs_layout_passes=False,
                                             use_tc_tiling_on_sc=False),
        scratch_types=[
            pltpu.VMEM_SHARED((ACC_R, D), jnp.float32),
            pltpu.VMEM_SHARED((ACC_R,), jnp.float32),
            pltpu.VMEM((ACC_R,), jnp.float32),
            pltpu.VMEM((ACC_R,), jnp.float32),
            pltpu.VMEM((SPAN_MAX, CH), jnp.int32),
            pltpu.VMEM((SPAN_MAX, CH), jnp.int32),
            pltpu.VMEM((CH,), jnp.float32),
            pltpu.VMEM((CH, D), jnp.float32),
            pltpu.VMEM((CH, D), jnp.float32),
            pltpu.SemaphoreType.DMA,
            pltpu.SemaphoreType.DMA,
            pltpu.SemaphoreType.DMA,
            pltpu.SemaphoreType.DMA,
        ],
    )
    def sc_kernel(zp1h, zp2h, zp3h, za1h, za2h, za3h, zsh, spth, sath, ssth,
                  pp_s, pp_d, ap_s, ap_d, pa_s, pa_d, aa_s, aa_d,
                  ps_s, ps_d, as_s, as_d, in_s, in_d,
                  accP, accA, accS, ssumP, ssumA, ssumS,
                  acc, ssum, sbuf, dbuf, src_i2, dst_i2, wbuf,
                  rows0, rows1, semg0, semg1, sems0, sems1):
        cid = lax.axis_index("c")
        sid = lax.axis_index("s")
        wid = sid * _NC + cid

        zero16 = jnp.zeros((_L,), jnp.float32)

        def zero_buf2d(buf):
            def body(r, c):
                for cc in range(D // _L):
                    buf[r, pl.ds(cc * _L, _L)] = zero16
                return c
            lax.fori_loop(0, CH, body, 0)

        def zero_wbuf():
            for j in range(CH // _L):
                wbuf[pl.ds(j * _L, _L)] = zero16

        def zero_acc(nrows):
            per = nrows // _NSUB
            base = sid * per
            off = 0
            while off < per:
                n = min(CH, per - off)
                pltpu.sync_copy(rows0.at[pl.ds(0, n)],
                                acc.at[pl.ds(base + off, n)])
                off += n

        def zero_ssum():
            per = ACC_R // _NSUB
            base = sid * per
            for j in range(per // CH):
                pltpu.sync_copy(wbuf, ssum.at[pl.ds(base + j * CH, CH)])

        def edge_w(r, j):
            sl = pl.ds(j * _L, _L)
            si = src_i2[r, sl]
            di = dst_i2[r, sl]
            sv = plsc.load_gather(sbuf, [si])
            dv = plsc.load_gather(dbuf, [di])
            e = sv + dv
            e = jnp.where(e > 0, e, 0.2 * e)
            return jnp.exp(e)

        def chunk_w(r):
            for j in range(CH // _L):
                wbuf[pl.ds(j * _L, _L)] = edge_w(r, j)
            pltpu.sync_copy(wbuf, ssum.at[dst_i2.at[r]], add=True)

        def scale_rows(rows):
            def rbody(r, c2):
                ridx = jnp.full((_L,), r, dtype=jnp.int32)
                av = plsc.load_gather(wbuf, [ridx])
                for cc in range(D // _L):
                    s2 = pl.ds(cc * _L, _L)
                    rows[r, s2] = rows[r, s2] * av
                return c2
            lax.fori_loop(0, CH, rbody, 0)

        def relation(wh_h, sv_h, srow, dv_h, drow, src_h, dst_h, nk,
                     acc_out, ssum_out, slot, out_rows):
            span = nk // _NW
            zero_wbuf()
            zero_buf2d(rows0)
            zero_ssum()
            zero_acc(ACC_R if out_rows > 1024 else 1024)
            pltpu.sync_copy(sv_h.at[srow], sbuf)
            pltpu.sync_copy(dv_h.at[drow], dbuf)
            base = wid * span
            pltpu.sync_copy(src_h.at[pl.ds(base, span)],
                            src_i2.at[pl.ds(0, span)])
            pltpu.sync_copy(dst_h.at[pl.ds(base, span)],
                            dst_i2.at[pl.ds(0, span)])
            plsc.subcore_barrier()

            def body(i, c):
                r0 = 2 * i
                r1 = 2 * i + 1
                g0 = pltpu.async_copy(wh_h.at[src_i2.at[r0]], rows0, semg0)
                g1 = pltpu.async_copy(wh_h.at[src_i2.at[r1]], rows1, semg1)
                chunk_w(r0)
                g0.wait()
                scale_rows(rows0)
                s0 = pltpu.async_copy(rows0, acc.at[dst_i2.at[r0]], sems0,
                                      add=True)
                chunk_w(r1)
                g1.wait()
                scale_rows(rows1)
                s1 = pltpu.async_copy(rows1, acc.at[dst_i2.at[r1]], sems1,
                                      add=True)
                s0.wait()
                s1.wait()
                return c
            lax.fori_loop(0, span // 2, body, 0)
            plsc.subcore_barrier()
            per = out_rows // _NSUB
            rbase = sid * per
            pltpu.sync_copy(acc.at[pl.ds(rbase, per)],
                            acc_out.at[slot, cid, pl.ds(rbase, per)])
            if ssum_out is not None:
                pltpu.sync_copy(ssum.at[pl.ds(rbase, per)],
                                ssum_out.at[slot, cid, pl.ds(rbase, per)])

        def in_relation(nk):
            zero_buf2d(rows0)
            zero_acc(1024)
            plsc.subcore_barrier()
            @pl.when(wid < nk)
            def _():
                pltpu.sync_copy(in_s.at[pl.ds(wid, 1)], src_i2.at[pl.ds(0, 1)])
                pltpu.sync_copy(in_d.at[pl.ds(wid, 1)], dst_i2.at[pl.ds(0, 1)])
                pltpu.async_copy(zsh.at[src_i2.at[0]], rows0, semg0).wait()
                pltpu.sync_copy(rows0, acc.at[dst_i2.at[0]], add=True)
            plsc.subcore_barrier()
            per = NS_N // _NSUB
            rbase = sid * per
            pltpu.sync_copy(acc.at[pl.ds(rbase, per)],
                            accS.at[2, cid, pl.ds(rbase, per)])

        relation(zp1h, spth, 0, spth, 3, pp_s, pp_d, nk_of["p2p"],
                 accP, ssumP, 0, ACC_R)
        relation(za1h, sath, 0, spth, 4, ap_s, ap_d, nk_of["a2p"],
                 accP, ssumP, 1, ACC_R)
        relation(zp2h, spth, 1, sath, 3, pa_s, pa_d, nk_of["p2a"],
                 accA, ssumA, 0, ACC_R)
        relation(za2h, sath, 1, sath, 4, aa_s, aa_d, nk_of["a2a"],
                 accA, ssumA, 1, ACC_R)
        relation(zp3h, spth, 2, ssth, 0, ps_s, ps_d, nk_of["p2s"],
                 accS, ssumS, 0, NS_N)
        relation(za3h, sath, 2, ssth, 1, as_s, as_d, nk_of["a2s"],
                 accS, ssumS, 1, NS_N)
        in_relation(nk_of["in"])

    e = edges
    return sc_kernel(zp1, zp2, zp3, za1, za2, za3, zs, spt, sat, sst,
                     e["p2p"][0], e["p2p"][1], e["a2p"][0], e["a2p"][1],
                     e["p2a"][0], e["p2a"][1], e["a2a"][0], e["a2a"][1],
                     e["p2s"][0], e["p2s"][1], e["a2s"][0], e["a2s"][1],
                     e["in"][0], e["in"][1])


def kernel(feat_P, feat_A, feat_state, W_P, b_P, W_A, b_A, W_p2p, b_p2p,
           W_p2a, b_p2a, W_a2p, b_a2p, W_a2a, b_a2a, W_p2s, b_p2s, W_a2s,
           b_a2s, W_in, b_in, a_p2p, a_p2a, a_a2p, a_a2a, a_p2s, a_a2s,
           edge_p2p, edge_p2a, edge_a2p, edge_a2a, edge_p2s, edge_a2s,
           edge_in):
    f32 = jnp.float32
    av_P = jnp.concatenate([
        a_p2p[:, :D], a_p2a[:, :D], a_p2s[:, :D],
        a_p2p[:, D:], a_a2p[:, D:], jnp.zeros((3, D), f32)], axis=0)
    av_A = jnp.concatenate([
        a_a2p[:, :D], a_a2a[:, :D], a_a2s[:, :D],
        a_p2a[:, D:], a_a2a[:, D:], jnp.zeros((3, D), f32)], axis=0)
    av_S = jnp.concatenate([a_p2s[:, D:], a_a2s[:, D:]], axis=0)
    bm_P = jnp.stack([b_p2p, b_p2a, b_p2s, b_P], axis=0)
    bm_A = jnp.stack([b_a2p, b_a2a, b_a2s, b_A], axis=0)

    zp1, zp2, zp3, SP = _node_transform(feat_P, W_p2p, W_p2a, W_p2s, W_P,
                                        bm_P, av_P, 400)
    za1, za2, za3, SA = _node_transform(feat_A, W_a2p, W_a2a, W_a2s, W_A,
                                        bm_A, av_A, 400)
    ZS, SS = _state_transform(feat_state, W_in, b_in, av_S)

    spt = jnp.concatenate([SP.T, jnp.zeros((8, ACC_R - NP_N), f32)], axis=1)
    sat = jnp.concatenate([SA.T, jnp.zeros((8, ACC_R - NA_N), f32)], axis=1)
    sst = jnp.concatenate([SS.T, jnp.zeros((8, ACC_R - NS_N), f32)], axis=1)

    edges = {
        "p2p": _pad_edge(edge_p2p.astype(jnp.int32), NP_N, 64),
        "a2p": _pad_edge(edge_a2p.astype(jnp.int32), NP_N, 64),
        "p2a": _pad_edge(edge_p2a.astype(jnp.int32), NA_N, 64),
        "a2a": _pad_edge(edge_a2a.astype(jnp.int32), NA_N, 64),
        "p2s": _pad_edge(edge_p2s.astype(jnp.int32), NS_N, 64),
        "a2s": _pad_edge(edge_a2s.astype(jnp.int32), NS_N, 64),
        "in": _pad_edge(edge_in.astype(jnp.int32), NS_N, 1),
    }

    aP, aA, aS, sP, sA, sS = _edge_sc(zp1, zp2, zp3, za1, za2, za3, ZS,
                                      spt, sat, sst, edges)
    hP = _combine_pa(aP, sP, NP_N, 512)
    hA = _combine_pa(aA, sA, NA_N, 512)
    hS = _combine_s(aS, sS)
    return (hP, hA, hS)


# speedup vs baseline: 1.1741x; 1.1741x over previous
import functools

import jax
import jax.numpy as jnp
from jax import lax
from jax.experimental import pallas as pl
from jax.experimental.pallas import tpu as pltpu
from jax.experimental.pallas import tpu_sc as plsc

NP_N = 10000
NA_N = 10000
NS_N = 512
D = 128

_NC = 2
_NSUB = 16
_L = 16
CH = 128
ACC_R = 10240


def _node_tc_kernel(x_ref, w1_ref, w2_ref, w3_ref, w0_ref, b_ref, av_ref,
                    z1_ref, z2_ref, z3_ref, s_ref):
    x = x_ref[...]
    dn = (((1,), (1,)), ((), ()))
    z1 = lax.dot_general(x, w1_ref[...], dn, preferred_element_type=jnp.float32)
    z2 = lax.dot_general(x, w2_ref[...], dn, preferred_element_type=jnp.float32)
    z3 = lax.dot_general(x, w3_ref[...], dn, preferred_element_type=jnp.float32)
    z0 = lax.dot_general(x, w0_ref[...], dn, preferred_element_type=jnp.float32)
    b = b_ref[...]
    z1 = z1 + b[0:1, :]
    z2 = z2 + b[1:2, :]
    z3 = z3 + b[2:3, :]
    z0 = z0 + b[3:4, :]
    av = av_ref[...]
    s1 = lax.dot_general(z1, av[0:1, :], dn, preferred_element_type=jnp.float32)
    s2 = lax.dot_general(z2, av[1:2, :], dn, preferred_element_type=jnp.float32)
    s3 = lax.dot_general(z3, av[2:3, :], dn, preferred_element_type=jnp.float32)
    d1 = lax.dot_general(z0, av[3:4, :], dn, preferred_element_type=jnp.float32)
    d2 = lax.dot_general(z0, av[4:5, :], dn, preferred_element_type=jnp.float32)
    z1_ref[...] = z1
    z2_ref[...] = z2
    z3_ref[...] = z3
    s_ref[...] = jnp.concatenate(
        [s1, s2, s3, d1, d2, jnp.zeros_like(s1), jnp.zeros_like(s1),
         jnp.zeros_like(s1)], axis=1)


def _node_transform(x, w1, w2, w3, w0, bmat, av, block_rows):
    n = x.shape[0]
    z_spec = pl.BlockSpec((block_rows, D), lambda i: (i, 0))
    return pl.pallas_call(
        _node_tc_kernel,
        grid=(n // block_rows,),
        in_specs=[
            pl.BlockSpec((block_rows, D), lambda i: (i, 0)),
            pl.BlockSpec((D, D), lambda i: (0, 0)),
            pl.BlockSpec((D, D), lambda i: (0, 0)),
            pl.BlockSpec((D, D), lambda i: (0, 0)),
            pl.BlockSpec((D, D), lambda i: (0, 0)),
            pl.BlockSpec((4, D), lambda i: (0, 0)),
            pl.BlockSpec((8, D), lambda i: (0, 0)),
        ],
        out_specs=[z_spec, z_spec, z_spec,
                   pl.BlockSpec((block_rows, 8), lambda i: (i, 0))],
        out_shape=[
            jax.ShapeDtypeStruct((n, D), jnp.float32),
            jax.ShapeDtypeStruct((n, D), jnp.float32),
            jax.ShapeDtypeStruct((n, D), jnp.float32),
            jax.ShapeDtypeStruct((n, 8), jnp.float32),
        ],
    )(x, w1, w2, w3, w0, bmat, av)


def _state_tc_kernel(x_ref, w0_ref, b_ref, av_ref, z_ref, s_ref):
    x = x_ref[...]
    dn = (((1,), (1,)), ((), ()))
    z0 = lax.dot_general(x, w0_ref[...], dn, preferred_element_type=jnp.float32)
    z0 = z0 + b_ref[...][0:1, :]
    av = av_ref[...]
    d1 = lax.dot_general(z0, av[0:1, :], dn, preferred_element_type=jnp.float32)
    d2 = lax.dot_general(z0, av[1:2, :], dn, preferred_element_type=jnp.float32)
    z_ref[...] = z0
    s_ref[...] = jnp.concatenate([d1, d2] + [jnp.zeros_like(d1)] * 6, axis=1)


def _state_transform(x, w0, b, av):
    n = x.shape[0]
    return pl.pallas_call(
        _state_tc_kernel,
        out_shape=[
            jax.ShapeDtypeStruct((n, D), jnp.float32),
            jax.ShapeDtypeStruct((n, 8), jnp.float32),
        ],
    )(x, w0, b.reshape(1, D), av)


def _combine_kernel(p_ref, o_ref):
    o_ref[...] = jnp.maximum(p_ref[0] + p_ref[1], 0.0)


def _combine(partials, n_out, block_rows):
    return pl.pallas_call(
        _combine_kernel,
        grid=(n_out // block_rows,),
        in_specs=[pl.BlockSpec((2, block_rows, D), lambda i: (0, i, 0))],
        out_specs=pl.BlockSpec((block_rows, D), lambda i: (i, 0)),
        out_shape=jax.ShapeDtypeStruct((n_out, D), jnp.float32),
    )(partials)


def _pad_edge(e32, pad_dst):
    n_e = e32.shape[1]
    pad = (-n_e) % CH
    if pad:
        src = jnp.concatenate([e32[0], jnp.zeros((pad,), jnp.int32)])
        dst = jnp.concatenate([e32[1], jnp.full((pad,), pad_dst, jnp.int32)])
    else:
        src, dst = e32[0], e32[1]
    return src, dst, (n_e + pad) // CH


def _edge_sc(zp1, zp2, zp3, za1, za2, za3, zs, spt, sat, sst, edges):
    nk_of = {r: edges[r][2] for r in edges}
    mesh = plsc.VectorSubcoreMesh(core_axis_name="c", subcore_axis_name="s",
                                  num_cores=_NC, num_subcores=_NSUB)

    @functools.partial(
        pl.kernel,
        out_type=[
            jax.ShapeDtypeStruct((_NC, ACC_R, D), jnp.float32),
            jax.ShapeDtypeStruct((_NC, ACC_R, D), jnp.float32),
            jax.ShapeDtypeStruct((_NC, NS_N, D), jnp.float32),
        ],
        mesh=mesh,
        compiler_params=pltpu.CompilerParams(needs_layout_passes=False),
        scratch_types=[
            pltpu.VMEM_SHARED((ACC_R, D), jnp.float32),
            pltpu.VMEM_SHARED((ACC_R,), jnp.float32),
            pltpu.VMEM((ACC_R,), jnp.float32),
            pltpu.VMEM((ACC_R,), jnp.float32),
            pltpu.VMEM((ACC_R,), jnp.float32),
            pltpu.VMEM((CH,), jnp.int32),
            pltpu.VMEM((CH,), jnp.int32),
            pltpu.VMEM((CH,), jnp.float32),
            pltpu.VMEM((CH,), jnp.float32),
            pltpu.VMEM((CH, D), jnp.float32),
            pltpu.SemaphoreType.DMA,
        ],
    )
    def sc_kernel(zp1h, zp2h, zp3h, za1h, za2h, za3h, zsh, spth, sath, ssth,
                  pp_s, pp_d, ap_s, ap_d, pa_s, pa_d, aa_s, aa_d,
                  ps_s, ps_d, as_s, as_d, in_s, in_d,
                  outP, outA, outS,
                  acc, ssum, sbuf, dbuf, sloc, src_ib, dst_ib, wbuf, abuf,
                  rows, sem):
        cid = lax.axis_index("c")
        sid = lax.axis_index("s")
        wid = sid * _NC + cid

        zero16 = jnp.zeros((_L,), jnp.float32)

        def zero_rows_buf():
            def body(r, c):
                for cc in range(D // _L):
                    rows[r, pl.ds(cc * _L, _L)] = zero16
                return c
            lax.fori_loop(0, CH, body, 0)

        def zero_wbuf():
            for j in range(CH // _L):
                wbuf[pl.ds(j * _L, _L)] = zero16

        def zero_acc(nrows):
            per = nrows // _NSUB
            base = sid * per
            off = 0
            while off < per:
                n = min(CH, per - off)
                pltpu.sync_copy(rows.at[pl.ds(0, n)],
                                acc.at[pl.ds(base + off, n)])
                off += n

        def zero_ssum():
            per = ACC_R // _NSUB
            base = sid * per
            for j in range(per // CH):
                pltpu.sync_copy(wbuf, ssum.at[pl.ds(base + j * CH, CH)])

        def edge_w(j):
            sl = pl.ds(j * _L, _L)
            si = src_ib[sl]
            di = dst_ib[sl]
            sv = plsc.load_gather(sbuf, [si])
            dv = plsc.load_gather(dbuf, [di])
            e = sv + dv
            e = jnp.where(e > 0, e, 0.2 * e)
            return jnp.exp(e), di

        def scalar_phase(src_h, dst_h, nk):
            it = -(-nk // _NSUB)
            def body(i, c):
                k = sid + _NSUB * i
                @pl.when(k < nk)
                def _():
                    off = pl.multiple_of(k * CH, CH)
                    pltpu.sync_copy(src_h.at[pl.ds(off, CH)], src_ib)
                    pltpu.sync_copy(dst_h.at[pl.ds(off, CH)], dst_ib)
                    for j in range(CH // _L):
                        w, _di = edge_w(j)
                        wbuf[pl.ds(j * _L, _L)] = w
                    pltpu.sync_copy(wbuf, ssum.at[dst_ib], add=True)
                return c
            lax.fori_loop(0, it, body, 0)

        def row_phase(wh_h, src_h, dst_h, nk, with_alpha):
            nw = _NC * _NSUB
            it = -(-nk // nw)
            def body(i, c):
                k = wid + nw * i
                @pl.when(k < nk)
                def _():
                    off = pl.multiple_of(k * CH, CH)
                    pltpu.sync_copy(src_h.at[pl.ds(off, CH)], src_ib)
                    pltpu.sync_copy(dst_h.at[pl.ds(off, CH)], dst_ib)
                    pltpu.async_copy(wh_h.at[src_ib], rows, sem).wait()
                    if with_alpha:
                        for j in range(CH // _L):
                            w, di = edge_w(j)
                            sg = plsc.load_gather(sloc, [di])
                            abuf[pl.ds(j * _L, _L)] = w / (sg + 1e-16)
                        def rbody(r, c2):
                            ridx = jnp.full((_L,), r, dtype=jnp.int32)
                            av = plsc.load_gather(abuf, [ridx])
                            for cc in range(D // _L):
                                s2 = pl.ds(cc * _L, _L)
                                rows[r, s2] = rows[r, s2] * av
                            return c2
                        lax.fori_loop(0, CH, rbody, 0)
                    pltpu.sync_copy(rows, acc.at[dst_ib], add=True)
                return c
            lax.fori_loop(0, it, body, 0)

        def relation(wh_h, sv_h, srow, dv_h, drow, src_h, dst_h, nk):
            zero_wbuf()
            zero_ssum()
            pltpu.sync_copy(sv_h.at[srow], sbuf)
            pltpu.sync_copy(dv_h.at[drow], dbuf)
            plsc.subcore_barrier()
            scalar_phase(src_h, dst_h, nk)
            plsc.subcore_barrier()
            pltpu.sync_copy(ssum, sloc)
            plsc.subcore_barrier()
            row_phase(wh_h, src_h, dst_h, nk, True)

        def dump(out_h, nrows):
            per = nrows // _NSUB
            base = sid * per
            pltpu.sync_copy(acc.at[pl.ds(base, per)],
                            out_h.at[cid, pl.ds(base, per)])

        zero_rows_buf()
        zero_acc(ACC_R)
        relation(zp1h, spth, 0, spth, 3, pp_s, pp_d, nk_of["p2p"])
        relation(za1h, sath, 0, spth, 4, ap_s, ap_d, nk_of["a2p"])
        plsc.subcore_barrier()
        dump(outP, ACC_R)
        plsc.subcore_barrier()

        zero_rows_buf()
        zero_acc(ACC_R)
        relation(zp2h, spth, 1, sath, 3, pa_s, pa_d, nk_of["p2a"])
        relation(za2h, sath, 1, sath, 4, aa_s, aa_d, nk_of["a2a"])
        plsc.subcore_barrier()
        dump(outA, ACC_R)
        plsc.subcore_barrier()

        zero_rows_buf()
        zero_acc(1024)
        relation(zp3h, spth, 2, ssth, 0, ps_s, ps_d, nk_of["p2s"])
        relation(za3h, sath, 2, ssth, 1, as_s, as_d, nk_of["a2s"])
        row_phase(zsh, in_s, in_d, nk_of["in"], False)
        plsc.subcore_barrier()
        dump(outS, NS_N)

    e = edges
    return sc_kernel(zp1, zp2, zp3, za1, za2, za3, zs, spt, sat, sst,
                     e["p2p"][0], e["p2p"][1], e["a2p"][0], e["a2p"][1],
                     e["p2a"][0], e["p2a"][1], e["a2a"][0], e["a2a"][1],
                     e["p2s"][0], e["p2s"][1], e["a2s"][0], e["a2s"][1],
                     e["in"][0], e["in"][1])


def kernel(feat_P, feat_A, feat_state, W_P, b_P, W_A, b_A, W_p2p, b_p2p,
           W_p2a, b_p2a, W_a2p, b_a2p, W_a2a, b_a2a, W_p2s, b_p2s, W_a2s,
           b_a2s, W_in, b_in, a_p2p, a_p2a, a_a2p, a_a2a, a_p2s, a_a2s,
           edge_p2p, edge_p2a, edge_a2p, edge_a2a, edge_p2s, edge_a2s,
           edge_in):
    f32 = jnp.float32
    av_P = jnp.concatenate([
        a_p2p[:, :D], a_p2a[:, :D], a_p2s[:, :D],
        a_p2p[:, D:], a_a2p[:, D:], jnp.zeros((3, D), f32)], axis=0)
    av_A = jnp.concatenate([
        a_a2p[:, :D], a_a2a[:, :D], a_a2s[:, :D],
        a_p2a[:, D:], a_a2a[:, D:], jnp.zeros((3, D), f32)], axis=0)
    av_S = jnp.concatenate([a_p2s[:, D:], a_a2s[:, D:]], axis=0)
    bm_P = jnp.stack([b_p2p, b_p2a, b_p2s, b_P], axis=0)
    bm_A = jnp.stack([b_a2p, b_a2a, b_a2s, b_A], axis=0)

    zp1, zp2, zp3, SP = _node_transform(feat_P, W_p2p, W_p2a, W_p2s, W_P,
                                        bm_P, av_P, 400)
    za1, za2, za3, SA = _node_transform(feat_A, W_a2p, W_a2a, W_a2s, W_A,
                                        bm_A, av_A, 400)
    ZS, SS = _state_transform(feat_state, W_in, b_in, av_S)

    spt = jnp.concatenate([SP.T, jnp.zeros((8, ACC_R - NP_N), f32)], axis=1)
    sat = jnp.concatenate([SA.T, jnp.zeros((8, ACC_R - NA_N), f32)], axis=1)
    sst = jnp.concatenate([SS.T, jnp.zeros((8, ACC_R - NS_N), f32)], axis=1)

    edges = {
        "p2p": _pad_edge(edge_p2p.astype(jnp.int32), NP_N),
        "a2p": _pad_edge(edge_a2p.astype(jnp.int32), NP_N),
        "p2a": _pad_edge(edge_p2a.astype(jnp.int32), NA_N),
        "a2a": _pad_edge(edge_a2a.astype(jnp.int32), NA_N),
        "p2s": _pad_edge(edge_p2s.astype(jnp.int32), NS_N),
        "a2s": _pad_edge(edge_a2s.astype(jnp.int32), NS_N),
        "in": _pad_edge(edge_in.astype(jnp.int32), NS_N),
    }

    pP, pA, pS = _edge_sc(zp1, zp2, zp3, za1, za2, za3, ZS,
                          spt, sat, sst, edges)
    hP = _combine(pP, NP_N, 400)
    hA = _combine(pA, NA_N, 400)
    hS = _combine(pS, NS_N, 512)
    return (hP, hA, hS)


# speedup vs baseline: 1.4371x; 1.2239x over previous
import functools

import jax
import jax.numpy as jnp
from jax import lax
from jax.experimental import pallas as pl
from jax.experimental.pallas import tpu as pltpu
from jax.experimental.pallas import tpu_sc as plsc

NP_N = 10000
NA_N = 10000
NS_N = 512
D = 128

_NC = 2
_NSUB = 16
_L = 16
CH = 128
ACC_R = 10240


def _node_tc_kernel(x_ref, w1_ref, w2_ref, w3_ref, w0_ref, b_ref, av_ref,
                    z1_ref, z2_ref, z3_ref, s_ref):
    x = x_ref[...]
    dn = (((1,), (1,)), ((), ()))
    z1 = lax.dot_general(x, w1_ref[...], dn, preferred_element_type=jnp.float32)
    z2 = lax.dot_general(x, w2_ref[...], dn, preferred_element_type=jnp.float32)
    z3 = lax.dot_general(x, w3_ref[...], dn, preferred_element_type=jnp.float32)
    z0 = lax.dot_general(x, w0_ref[...], dn, preferred_element_type=jnp.float32)
    b = b_ref[...]
    z1 = z1 + b[0:1, :]
    z2 = z2 + b[1:2, :]
    z3 = z3 + b[2:3, :]
    z0 = z0 + b[3:4, :]
    av = av_ref[...]
    s1 = lax.dot_general(z1, av[0:1, :], dn, preferred_element_type=jnp.float32)
    s2 = lax.dot_general(z2, av[1:2, :], dn, preferred_element_type=jnp.float32)
    s3 = lax.dot_general(z3, av[2:3, :], dn, preferred_element_type=jnp.float32)
    d1 = lax.dot_general(z0, av[3:4, :], dn, preferred_element_type=jnp.float32)
    d2 = lax.dot_general(z0, av[4:5, :], dn, preferred_element_type=jnp.float32)
    z1_ref[...] = z1
    z2_ref[...] = z2
    z3_ref[...] = z3
    s_ref[...] = jnp.concatenate(
        [s1, s2, s3, d1, d2, jnp.zeros_like(s1), jnp.zeros_like(s1),
         jnp.zeros_like(s1)], axis=1)


def _node_transform(x, w1, w2, w3, w0, bmat, av, block_rows):
    n = x.shape[0]
    z_spec = pl.BlockSpec((block_rows, D), lambda i: (i, 0))
    return pl.pallas_call(
        _node_tc_kernel,
        grid=(n // block_rows,),
        in_specs=[
            pl.BlockSpec((block_rows, D), lambda i: (i, 0)),
            pl.BlockSpec((D, D), lambda i: (0, 0)),
            pl.BlockSpec((D, D), lambda i: (0, 0)),
            pl.BlockSpec((D, D), lambda i: (0, 0)),
            pl.BlockSpec((D, D), lambda i: (0, 0)),
            pl.BlockSpec((4, D), lambda i: (0, 0)),
            pl.BlockSpec((8, D), lambda i: (0, 0)),
        ],
        out_specs=[z_spec, z_spec, z_spec,
                   pl.BlockSpec((block_rows, 8), lambda i: (i, 0))],
        out_shape=[
            jax.ShapeDtypeStruct((n, D), jnp.float32),
            jax.ShapeDtypeStruct((n, D), jnp.float32),
            jax.ShapeDtypeStruct((n, D), jnp.float32),
            jax.ShapeDtypeStruct((n, 8), jnp.float32),
        ],
    )(x, w1, w2, w3, w0, bmat, av)


def _state_tc_kernel(x_ref, w0_ref, b_ref, av_ref, z_ref, s_ref):
    x = x_ref[...]
    dn = (((1,), (1,)), ((), ()))
    z0 = lax.dot_general(x, w0_ref[...], dn, preferred_element_type=jnp.float32)
    z0 = z0 + b_ref[...][0:1, :]
    av = av_ref[...]
    d1 = lax.dot_general(z0, av[0:1, :], dn, preferred_element_type=jnp.float32)
    d2 = lax.dot_general(z0, av[1:2, :], dn, preferred_element_type=jnp.float32)
    z_ref[...] = z0
    s_ref[...] = jnp.concatenate([d1, d2] + [jnp.zeros_like(d1)] * 6, axis=1)


def _state_transform(x, w0, b, av):
    n = x.shape[0]
    return pl.pallas_call(
        _state_tc_kernel,
        out_shape=[
            jax.ShapeDtypeStruct((n, D), jnp.float32),
            jax.ShapeDtypeStruct((n, 8), jnp.float32),
        ],
    )(x, w0, b.reshape(1, D), av)


def _combine_kernel(p_ref, o_ref):
    o_ref[...] = jnp.maximum(p_ref[0] + p_ref[1], 0.0)


def _combine(partials, n_out, block_rows):
    return pl.pallas_call(
        _combine_kernel,
        grid=(n_out // block_rows,),
        in_specs=[pl.BlockSpec((2, block_rows, D), lambda i: (0, i, 0))],
        out_specs=pl.BlockSpec((block_rows, D), lambda i: (i, 0)),
        out_shape=jax.ShapeDtypeStruct((n_out, D), jnp.float32),
    )(partials)


def _pad_edge(e32, pad_dst):
    n_e = e32.shape[1]
    pad = (-n_e) % CH
    if pad:
        src = jnp.concatenate([e32[0], jnp.zeros((pad,), jnp.int32)])
        dst = jnp.concatenate([e32[1], jnp.full((pad,), pad_dst, jnp.int32)])
    else:
        src, dst = e32[0], e32[1]
    return src, dst, (n_e + pad) // CH


def _edge_sc(zp1, zp2, zp3, za1, za2, za3, zs, spt, sat, sst, edges):
    nk_of = {r: edges[r][2] for r in edges}
    mesh = plsc.VectorSubcoreMesh(core_axis_name="c", subcore_axis_name="s",
                                  num_cores=_NC, num_subcores=_NSUB)

    @functools.partial(
        pl.kernel,
        out_type=[
            jax.ShapeDtypeStruct((_NC, ACC_R, D), jnp.float32),
            jax.ShapeDtypeStruct((_NC, ACC_R, D), jnp.float32),
            jax.ShapeDtypeStruct((_NC, NS_N, D), jnp.float32),
        ],
        mesh=mesh,
        compiler_params=pltpu.CompilerParams(needs_layout_passes=False),
        scratch_types=[
            pltpu.VMEM_SHARED((ACC_R, D), jnp.float32),
            pltpu.VMEM_SHARED((ACC_R,), jnp.float32),
            pltpu.VMEM((ACC_R,), jnp.float32),
            pltpu.VMEM((ACC_R,), jnp.float32),
            pltpu.VMEM((ACC_R,), jnp.float32),
            pltpu.VMEM((CH,), jnp.int32),
            pltpu.VMEM((CH,), jnp.int32),
            pltpu.VMEM((CH,), jnp.float32),
            pltpu.VMEM((CH,), jnp.float32),
            pltpu.VMEM((CH, D), jnp.float32),
            pltpu.SemaphoreType.DMA,
            pltpu.SemaphoreType.DMA,
            pltpu.SemaphoreType.DMA,
        ],
    )
    def sc_kernel(zp1h, zp2h, zp3h, za1h, za2h, za3h, zsh, spth, sath, ssth,
                  pp_s, pp_d, ap_s, ap_d, pa_s, pa_d, aa_s, aa_d,
                  ps_s, ps_d, as_s, as_d, in_s, in_d,
                  outP, outA, outS,
                  acc, ssum, sbuf, dbuf, sloc, src_ib, dst_ib, wbuf, abuf,
                  rows, sem, sem2, sem3):
        cid = lax.axis_index("c")
        sid = lax.axis_index("s")
        wid = sid * _NC + cid

        zero16 = jnp.zeros((_L,), jnp.float32)

        def zero_rows_buf():
            def body(r, c):
                for cc in range(D // _L):
                    rows[r, pl.ds(cc * _L, _L)] = zero16
                return c
            lax.fori_loop(0, CH, body, 0)

        def zero_wbuf():
            for j in range(CH // _L):
                wbuf[pl.ds(j * _L, _L)] = zero16

        def zero_acc(nrows):
            per = nrows // _NSUB
            base = sid * per
            off = 0
            while off < per:
                n = min(CH, per - off)
                pltpu.sync_copy(rows.at[pl.ds(0, n)],
                                acc.at[pl.ds(base + off, n)])
                off += n

        def zero_ssum():
            per = ACC_R // _NSUB
            base = sid * per
            for j in range(per // CH):
                pltpu.sync_copy(wbuf, ssum.at[pl.ds(base + j * CH, CH)])

        def edge_w(j):
            sl = pl.ds(j * _L, _L)
            si = src_ib[sl]
            di = dst_ib[sl]
            sv = plsc.load_gather(sbuf, [si])
            dv = plsc.load_gather(dbuf, [di])
            e = sv + dv
            e = jnp.where(e > 0, e, 0.2 * e)
            return jnp.exp(e), di

        def scalar_phase(src_h, dst_h, nk):
            it = -(-nk // _NSUB)
            def body(i, c):
                k = sid + _NSUB * i
                @pl.when(k < nk)
                def _():
                    off = pl.multiple_of(k * CH, CH)
                    c1 = pltpu.async_copy(src_h.at[pl.ds(off, CH)], src_ib,
                                          sem2)
                    c2 = pltpu.async_copy(dst_h.at[pl.ds(off, CH)], dst_ib,
                                          sem3)
                    c1.wait()
                    c2.wait()
                    for j in range(CH // _L):
                        w, _di = edge_w(j)
                        wbuf[pl.ds(j * _L, _L)] = w
                    pltpu.sync_copy(wbuf, ssum.at[dst_ib], add=True)
                return c
            lax.fori_loop(0, it, body, 0)

        def row_phase(wh_h, src_h, dst_h, nk, with_alpha):
            nw = _NC * _NSUB
            it = -(-nk // nw)
            def body(i, c):
                k = wid + nw * i
                @pl.when(k < nk)
                def _():
                    off = pl.multiple_of(k * CH, CH)
                    c1 = pltpu.async_copy(src_h.at[pl.ds(off, CH)], src_ib,
                                          sem2)
                    c2 = pltpu.async_copy(dst_h.at[pl.ds(off, CH)], dst_ib,
                                          sem3)
                    c1.wait()
                    c2.wait()
                    g = pltpu.async_copy(wh_h.at[src_ib], rows, sem)
                    if with_alpha:
                        for j in range(CH // _L):
                            w, di = edge_w(j)
                            sg = plsc.load_gather(sloc, [di])
                            abuf[pl.ds(j * _L, _L)] = w / (sg + 1e-16)
                        g.wait()
                        def rbody(r, c2):
                            ridx = jnp.full((_L,), r, dtype=jnp.int32)
                            av = plsc.load_gather(abuf, [ridx])
                            for cc in range(D // _L):
                                s2 = pl.ds(cc * _L, _L)
                                rows[r, s2] = rows[r, s2] * av
                            return c2
                        lax.fori_loop(0, CH, rbody, 0)
                    else:
                        g.wait()
                    pltpu.sync_copy(rows, acc.at[dst_ib], add=True)
                return c
            lax.fori_loop(0, it, body, 0)

        def relation(wh_h, sv_h, srow, dv_h, drow, src_h, dst_h, nk):
            zero_wbuf()
            zero_ssum()
            pltpu.sync_copy(sv_h.at[srow], sbuf)
            pltpu.sync_copy(dv_h.at[drow], dbuf)
            plsc.subcore_barrier()
            scalar_phase(src_h, dst_h, nk)
            plsc.subcore_barrier()
            pltpu.sync_copy(ssum, sloc)
            plsc.subcore_barrier()
            row_phase(wh_h, src_h, dst_h, nk, True)

        def dump(out_h, nrows):
            per = nrows // _NSUB
            base = sid * per
            pltpu.sync_copy(acc.at[pl.ds(base, per)],
                            out_h.at[cid, pl.ds(base, per)])

        zero_rows_buf()
        zero_acc(ACC_R)
        relation(zp1h, spth, 0, spth, 3, pp_s, pp_d, nk_of["p2p"])
        relation(za1h, sath, 0, spth, 4, ap_s, ap_d, nk_of["a2p"])
        plsc.subcore_barrier()
        dump(outP, ACC_R)
        plsc.subcore_barrier()

        zero_rows_buf()
        zero_acc(ACC_R)
        relation(zp2h, spth, 1, sath, 3, pa_s, pa_d, nk_of["p2a"])
        relation(za2h, sath, 1, sath, 4, aa_s, aa_d, nk_of["a2a"])
        plsc.subcore_barrier()
        dump(outA, ACC_R)
        plsc.subcore_barrier()

        zero_rows_buf()
        zero_acc(1024)
        relation(zp3h, spth, 2, ssth, 0, ps_s, ps_d, nk_of["p2s"])
        relation(za3h, sath, 2, ssth, 1, as_s, as_d, nk_of["a2s"])
        row_phase(zsh, in_s, in_d, nk_of["in"], False)
        plsc.subcore_barrier()
        dump(outS, NS_N)

    e = edges
    return sc_kernel(zp1, zp2, zp3, za1, za2, za3, zs, spt, sat, sst,
                     e["p2p"][0], e["p2p"][1], e["a2p"][0], e["a2p"][1],
                     e["p2a"][0], e["p2a"][1], e["a2a"][0], e["a2a"][1],
                     e["p2s"][0], e["p2s"][1], e["a2s"][0], e["a2s"][1],
                     e["in"][0], e["in"][1])


def kernel(feat_P, feat_A, feat_state, W_P, b_P, W_A, b_A, W_p2p, b_p2p,
           W_p2a, b_p2a, W_a2p, b_a2p, W_a2a, b_a2a, W_p2s, b_p2s, W_a2s,
           b_a2s, W_in, b_in, a_p2p, a_p2a, a_a2p, a_a2a, a_p2s, a_a2s,
           edge_p2p, edge_p2a, edge_a2p, edge_a2a, edge_p2s, edge_a2s,
           edge_in):
    f32 = jnp.float32
    av_P = jnp.concatenate([
        a_p2p[:, :D], a_p2a[:, :D], a_p2s[:, :D],
        a_p2p[:, D:], a_a2p[:, D:], jnp.zeros((3, D), f32)], axis=0)
    av_A = jnp.concatenate([
        a_a2p[:, :D], a_a2a[:, :D], a_a2s[:, :D],
        a_p2a[:, D:], a_a2a[:, D:], jnp.zeros((3, D), f32)], axis=0)
    av_S = jnp.concatenate([a_p2s[:, D:], a_a2s[:, D:]], axis=0)
    bm_P = jnp.stack([b_p2p, b_p2a, b_p2s, b_P], axis=0)
    bm_A = jnp.stack([b_a2p, b_a2a, b_a2s, b_A], axis=0)

    zp1, zp2, zp3, SP = _node_transform(feat_P, W_p2p, W_p2a, W_p2s, W_P,
                                        bm_P, av_P, 400)
    za1, za2, za3, SA = _node_transform(feat_A, W_a2p, W_a2a, W_a2s, W_A,
                                        bm_A, av_A, 400)
    ZS, SS = _state_transform(feat_state, W_in, b_in, av_S)

    spt = jnp.concatenate([SP.T, jnp.zeros((8, ACC_R - NP_N), f32)], axis=1)
    sat = jnp.concatenate([SA.T, jnp.zeros((8, ACC_R - NA_N), f32)], axis=1)
    sst = jnp.concatenate([SS.T, jnp.zeros((8, ACC_R - NS_N), f32)], axis=1)

    edges = {
        "p2p": _pad_edge(edge_p2p.astype(jnp.int32), NP_N),
        "a2p": _pad_edge(edge_a2p.astype(jnp.int32), NP_N),
        "p2a": _pad_edge(edge_p2a.astype(jnp.int32), NA_N),
        "a2a": _pad_edge(edge_a2a.astype(jnp.int32), NA_N),
        "p2s": _pad_edge(edge_p2s.astype(jnp.int32), NS_N),
        "a2s": _pad_edge(edge_a2s.astype(jnp.int32), NS_N),
        "in": _pad_edge(edge_in.astype(jnp.int32), NS_N),
    }

    pP, pA, pS = _edge_sc(zp1, zp2, zp3, za1, za2, za3, ZS,
                          spt, sat, sst, edges)
    hP = _combine(pP, NP_N, 400)
    hA = _combine(pA, NA_N, 400)
    hS = _combine(pS, NS_N, 512)
    return (hP, hA, hS)
